# baseline (speedup 1.0000x reference)
import jax
import jax.numpy as jnp
from jax import lax
from jax.experimental import pallas as pl
from jax.experimental.pallas import tpu as pltpu
from jax.experimental.pallas import tpu_sc as plsc


def _pos_plane_body(h_hbm, w_hbm, s_hbm, hrow_v, wrows_v, obuf_v):
    c = lax.axis_index("c")
    s = lax.axis_index("s")
    wid = s * 2 + c
    W = wrows_v.shape[0]
    D = hrow_v.shape[1]
    pltpu.sync_copy(h_hbm.at[pl.ds(wid, 1)], hrow_v)
    pltpu.sync_copy(w_hbm.at[pl.ds(0, W)], wrows_v)

    def body_w(w, carry):
        def body_c(cc, carry2):
            off = pl.multiple_of(cc * 16, 16)
            vh = hrow_v[0, pl.ds(off, 16)]
            vw = wrows_v[w, pl.ds(off, 16)]
            obuf_v[w, pl.ds(off, 16)] = vh + vw
            return carry2
        return lax.fori_loop(0, D // 16, body_c, carry)

    lax.fori_loop(0, W, body_w, 0)
    pltpu.sync_copy(obuf_v, s_hbm.at[pl.ds(wid * W, W)])


def _pos_plane(h_table, w_table, H, W):
    D = h_table.shape[1]
    mesh = plsc.VectorSubcoreMesh(core_axis_name="c", subcore_axis_name="s")
    kern = pl.kernel(
        _pos_plane_body,
        out_type=jax.ShapeDtypeStruct((H * W, D), jnp.float32),
        mesh=mesh,
        scratch_types=[
            pltpu.VMEM((1, D), jnp.float32),
            pltpu.VMEM((W, D), jnp.float32),
            pltpu.VMEM((W, D), jnp.float32),
        ],
    )
    return kern(h_table, w_table)


def _add_kernel(x_ref, s_ref, o_ref):
    o_ref[...] = x_ref[...] + s_ref[...]


def kernel(x, h_table, w_table):
    B, H, W, D = x.shape
    s = _pos_plane(h_table, w_table, H, W).reshape(1, H, W, D)
    return pl.pallas_call(
        _add_kernel,
        grid=(B,),
        in_specs=[
            pl.BlockSpec((1, H, W, D), lambda b: (b, 0, 0, 0)),
            pl.BlockSpec((1, H, W, D), lambda b: (0, 0, 0, 0)),
        ],
        out_specs=pl.BlockSpec((1, H, W, D), lambda b: (b, 0, 0, 0)),
        out_shape=jax.ShapeDtypeStruct((B, H, W, D), x.dtype),
    )(x, s)


# speedup vs baseline: 1.2470x; 1.2470x over previous
import jax
import jax.numpy as jnp
from jax.experimental import pallas as pl


def _add_pos_kernel(x_ref, h_ref, w_ref, o_ref):
    h = h_ref[...][:, :, None, :]
    w = w_ref[...][:, None, :, :]
    o_ref[...] = x_ref[...] + h + w


def kernel(x, h_table, w_table):
    B, H, W, D = x.shape
    HB = H // 2
    return pl.pallas_call(
        _add_pos_kernel,
        grid=(B, 2),
        in_specs=[
            pl.BlockSpec((1, HB, W, D), lambda b, i: (b, i, 0, 0)),
            pl.BlockSpec((1, HB, D), lambda b, i: (0, i, 0)),
            pl.BlockSpec((1, W, D), lambda b, i: (0, 0, 0)),
        ],
        out_specs=pl.BlockSpec((1, HB, W, D), lambda b, i: (b, i, 0, 0)),
        out_shape=jax.ShapeDtypeStruct((B, H, W, D), x.dtype),
    )(x, h_table[None], w_table[None])


# speedup vs baseline: 1.5672x; 1.2568x over previous
import jax
import jax.numpy as jnp
from jax.experimental import pallas as pl


def _add_pos_kernel(x_ref, h_ref, w_ref, o_ref):
    h = h_ref[...][:, :, None, :]
    w = w_ref[...][:, None, :, :]
    o_ref[...] = x_ref[...] + h + w


def kernel(x, h_table, w_table):
    B, H, W, D = x.shape
    BB = 2
    return pl.pallas_call(
        _add_pos_kernel,
        grid=(B // BB,),
        in_specs=[
            pl.BlockSpec((BB, H, W, D), lambda b: (b, 0, 0, 0)),
            pl.BlockSpec((1, H, D), lambda b: (0, 0, 0)),
            pl.BlockSpec((1, W, D), lambda b: (0, 0, 0)),
        ],
        out_specs=pl.BlockSpec((BB, H, W, D), lambda b: (b, 0, 0, 0)),
        out_shape=jax.ShapeDtypeStruct((B, H, W, D), x.dtype),
    )(x, h_table[None], w_table[None])


# speedup vs baseline: 1.5997x; 1.0207x over previous
import jax
import jax.numpy as jnp
from jax.experimental import pallas as pl


def _add_pos_kernel(x_ref, h_ref, w_ref, o_ref):
    h = h_ref[...][:, :, None, :]
    w = w_ref[...][:, None, :, :]
    o_ref[...] = x_ref[...] + h + w


def kernel(x, h_table, w_table):
    B, H, W, D = x.shape
    BB = 4
    return pl.pallas_call(
        _add_pos_kernel,
        grid=(B // BB,),
        in_specs=[
            pl.BlockSpec((BB, H, W, D), lambda b: (b, 0, 0, 0)),
            pl.BlockSpec((1, H, D), lambda b: (0, 0, 0)),
            pl.BlockSpec((1, W, D), lambda b: (0, 0, 0)),
        ],
        out_specs=pl.BlockSpec((BB, H, W, D), lambda b: (b, 0, 0, 0)),
        out_shape=jax.ShapeDtypeStruct((B, H, W, D), x.dtype),
    )(x, h_table[None], w_table[None])
